# baseline (speedup 1.0000x reference)
import functools

import jax
import jax.numpy as jnp
from jax import lax
from jax.experimental import pallas as pl
from jax.experimental.pallas import tpu as pltpu
from jax.experimental.pallas import tpu_sc as plsc

_N_KF = 8
_N_MP = 32
_FX = 320.0
_FY = 320.0
_CX = 320.0
_CY = 240.0
_L = 16


@functools.lru_cache(maxsize=None)
def _build_sc_call(m: int):
    info = plsc.get_sparse_core_info()
    nc, ns = info.num_cores, info.num_subcores
    nw = nc * ns
    assert m % (nw * _L) == 0
    ch = m // nw
    n_it = ch // _L

    mesh = plsc.VectorSubcoreMesh(core_axis_name="c", subcore_axis_name="s")

    @functools.partial(
        pl.kernel,
        out_type=jax.ShapeDtypeStruct((2 * m,), jnp.float32),
        mesh=mesh,
        scratch_types=[
            pltpu.VMEM((ch,), jnp.int32),
            pltpu.VMEM((ch,), jnp.int32),
            pltpu.VMEM((2 * ch,), jnp.float32),
            pltpu.VMEM((_N_KF * _N_MP,), jnp.float32),
            pltpu.VMEM((_N_KF * _N_MP,), jnp.float32),
            pltpu.VMEM((_L,), jnp.int32),
            pltpu.VMEM((_N_MP,), jnp.int32),
            pltpu.VMEM((3 * _N_MP,), jnp.float32),
            pltpu.VMEM((_N_KF * 16,), jnp.float32),
            pltpu.VMEM((_L,), jnp.int32),
            pltpu.VMEM((_N_MP,), jnp.int32),
        ],
        compiler_params=pltpu.CompilerParams(needs_layout_passes=False),
    )
    def sc_call(kf_hbm, mp_hbm, tmpt_hbm, tkf_hbm, idxkf_hbm, idxmp_hbm,
                out_hbm, inkf, inmp, outb, tblx, tbly,
                poskf, posmp, tmpt, tkf, idxkf, idxmp):
        wid = lax.axis_index("s") * nc + lax.axis_index("c")
        base = wid * jnp.int32(ch)

        pltpu.sync_copy(kf_hbm.at[pl.ds(base, ch)], inkf)
        pltpu.sync_copy(mp_hbm.at[pl.ds(base, ch)], inmp)
        pltpu.sync_copy(tmpt_hbm, tmpt)
        pltpu.sync_copy(tkf_hbm, tkf)
        pltpu.sync_copy(idxkf_hbm, idxkf)
        pltpu.sync_copy(idxmp_hbm, idxmp)

        iota = lax.iota(jnp.int32, _L)

        plsc.store_scatter(poskf, [idxkf[...]], iota)
        plsc.store_scatter(posmp, [idxmp[pl.ds(0, _L)]], iota)
        plsc.store_scatter(posmp, [idxmp[pl.ds(_L, _L)]], iota + _L)

        xp, yp, zp = [], [], []
        for h in range(2):
            pos = posmp[pl.ds(h * _L, _L)]
            xp.append(plsc.load_gather(tmpt, [pos]))
            yp.append(plsc.load_gather(tmpt, [pos + _N_MP]))
            zp.append(plsc.load_gather(tmpt, [pos + 2 * _N_MP]))

        pk = poskf[...]
        for v in range(_N_KF):
            row = tkf[pl.ds(pk[v] * jnp.int32(16), 16)]
            a00 = row[0]; a01 = row[1]; a02 = row[2]; a03 = row[3]
            a10 = row[4]; a11 = row[5]; a12 = row[6]; a13 = row[7]
            a20 = row[8]; a21 = row[9]; a22 = row[10]; a23 = row[11]
            for h in range(2):
                r0 = a00 * xp[h] + a01 * yp[h] + a02 * zp[h] + a03
                r1 = a10 * xp[h] + a11 * yp[h] + a12 * zp[h] + a13
                r2 = a20 * xp[h] + a21 * yp[h] + a22 * zp[h] + a23
                inv = 1.0 / r2
                tb = v * _N_MP + h * _L
                tblx[pl.ds(tb, _L)] = r0 * inv * _FX + _CX
                tbly[pl.ds(tb, _L)] = r1 * inv * _FY + _CY

        def body(i, carry):
            off = i * jnp.int32(_L)
            off2 = (jnp.right_shift(i, 3) * jnp.int32(256)
                    + jnp.bitwise_and(i, 7) * jnp.int32(_L))
            idx = jnp.left_shift(inkf[pl.ds(off, _L)], 5) + inmp[pl.ds(off, _L)]
            outb[pl.ds(off2, _L)] = plsc.load_gather(tblx, [idx])
            outb[pl.ds(off2 + jnp.int32(128), _L)] = plsc.load_gather(tbly, [idx])
            return carry

        lax.fori_loop(jnp.int32(0), jnp.int32(n_it), body, jnp.int32(0))

        pltpu.sync_copy(outb, out_hbm.at[pl.ds(2 * base, 2 * ch)])

    return sc_call


def kernel(measurements, tMP, tKF, idxMP, idxKF):
    m = measurements.shape[0]
    out_dtype = jnp.promote_types(tMP.dtype, tKF.dtype)
    kf = measurements[:, 0].astype(jnp.int32)
    mp = measurements[:, 1].astype(jnp.int32)
    tmpt = tMP.astype(jnp.float32).T.reshape(-1)
    tkf = tKF.astype(jnp.float32).reshape(-1)
    idxkf = jnp.concatenate(
        [idxKF.astype(jnp.int32),
         jnp.arange(_N_KF, _L, dtype=jnp.int32)])
    idxmp = idxMP.astype(jnp.int32)
    out = _build_sc_call(m)(kf, mp, tmpt, tkf, idxkf, idxmp)
    out = out.reshape(m // 128, 2, 128).transpose(0, 2, 1).reshape(m, 2)
    return out.astype(out_dtype)


# speedup vs baseline: 1.0274x; 1.0274x over previous
import functools

import jax
import jax.numpy as jnp
from jax import lax
from jax.experimental import pallas as pl
from jax.experimental.pallas import tpu as pltpu
from jax.experimental.pallas import tpu_sc as plsc

_N_KF = 8
_N_MP = 32
_FX = 320.0
_FY = 320.0
_CX = 320.0
_CY = 240.0
_L = 16


@functools.lru_cache(maxsize=None)
def _build_sc_call(m: int):
    info = plsc.get_sparse_core_info()
    nc, ns = info.num_cores, info.num_subcores
    nw = nc * ns
    assert m % (nw * _L) == 0
    ch = m // nw
    n_it = ch // _L
    hf = ch // 2

    mesh = plsc.VectorSubcoreMesh(core_axis_name="c", subcore_axis_name="s")

    @functools.partial(
        pl.kernel,
        out_type=jax.ShapeDtypeStruct((2 * m,), jnp.float32),
        mesh=mesh,
        scratch_types=[
            pltpu.VMEM((ch,), jnp.int32),
            pltpu.VMEM((ch,), jnp.int32),
            pltpu.VMEM((2 * ch,), jnp.float32),
            pltpu.VMEM((_N_KF * _N_MP,), jnp.float32),
            pltpu.VMEM((_N_KF * _N_MP,), jnp.float32),
            pltpu.VMEM((_L,), jnp.int32),
            pltpu.VMEM((_N_MP,), jnp.int32),
            pltpu.VMEM((3 * _N_MP,), jnp.float32),
            pltpu.VMEM((_N_KF * 16,), jnp.float32),
            pltpu.VMEM((_L,), jnp.int32),
            pltpu.VMEM((_N_MP,), jnp.int32),
            pltpu.SemaphoreType.DMA,
            pltpu.SemaphoreType.DMA,
            pltpu.SemaphoreType.DMA,
        ],
        compiler_params=pltpu.CompilerParams(needs_layout_passes=False),
    )
    def sc_call(kf_hbm, mp_hbm, tmpt_hbm, tkf_hbm, idxkf_hbm, idxmp_hbm,
                out_hbm, inkf, inmp, outb, tblx, tbly,
                poskf, posmp, tmpt, tkf, idxkf, idxmp,
                sem_in0, sem_in1, sem_out):
        wid = lax.axis_index("s") * nc + lax.axis_index("c")
        base = wid * jnp.int32(ch)

        in0a = pltpu.async_copy(kf_hbm.at[pl.ds(base, hf)],
                                inkf.at[pl.ds(0, hf)], sem_in0)
        in0b = pltpu.async_copy(mp_hbm.at[pl.ds(base, hf)],
                                inmp.at[pl.ds(0, hf)], sem_in0)
        in1a = pltpu.async_copy(kf_hbm.at[pl.ds(base + jnp.int32(hf), hf)],
                                inkf.at[pl.ds(hf, hf)], sem_in1)
        in1b = pltpu.async_copy(mp_hbm.at[pl.ds(base + jnp.int32(hf), hf)],
                                inmp.at[pl.ds(hf, hf)], sem_in1)
        pltpu.sync_copy(tmpt_hbm, tmpt)
        pltpu.sync_copy(tkf_hbm, tkf)
        pltpu.sync_copy(idxkf_hbm, idxkf)
        pltpu.sync_copy(idxmp_hbm, idxmp)

        iota = lax.iota(jnp.int32, _L)

        plsc.store_scatter(poskf, [idxkf[...]], iota)
        plsc.store_scatter(posmp, [idxmp[pl.ds(0, _L)]], iota)
        plsc.store_scatter(posmp, [idxmp[pl.ds(_L, _L)]], iota + _L)

        xp, yp, zp = [], [], []
        for h in range(2):
            pos = posmp[pl.ds(h * _L, _L)]
            xp.append(plsc.load_gather(tmpt, [pos]))
            yp.append(plsc.load_gather(tmpt, [pos + _N_MP]))
            zp.append(plsc.load_gather(tmpt, [pos + 2 * _N_MP]))

        pk = poskf[...]
        for v in range(_N_KF):
            row = tkf[pl.ds(pk[v] * jnp.int32(16), 16)]
            a00 = row[0]; a01 = row[1]; a02 = row[2]; a03 = row[3]
            a10 = row[4]; a11 = row[5]; a12 = row[6]; a13 = row[7]
            a20 = row[8]; a21 = row[9]; a22 = row[10]; a23 = row[11]
            for h in range(2):
                r0 = a00 * xp[h] + a01 * yp[h] + a02 * zp[h] + a03
                r1 = a10 * xp[h] + a11 * yp[h] + a12 * zp[h] + a13
                r2 = a20 * xp[h] + a21 * yp[h] + a22 * zp[h] + a23
                inv = 1.0 / r2
                tb = v * _N_MP + h * _L
                tblx[pl.ds(tb, _L)] = r0 * inv * _FX + _CX
                tbly[pl.ds(tb, _L)] = r1 * inv * _FY + _CY

        def body(i, carry):
            off = i * jnp.int32(_L)
            off2 = (jnp.right_shift(i, 3) * jnp.int32(256)
                    + jnp.bitwise_and(i, 7) * jnp.int32(_L))
            idx = jnp.left_shift(inkf[pl.ds(off, _L)], 5) + inmp[pl.ds(off, _L)]
            outb[pl.ds(off2, _L)] = plsc.load_gather(tblx, [idx])
            outb[pl.ds(off2 + jnp.int32(128), _L)] = plsc.load_gather(tbly, [idx])
            return carry

        in0a.wait()
        in0b.wait()
        lax.fori_loop(jnp.int32(0), jnp.int32(n_it // 2), body, jnp.int32(0))
        out0 = pltpu.async_copy(outb.at[pl.ds(0, ch)],
                                out_hbm.at[pl.ds(2 * base, ch)], sem_out)
        in1a.wait()
        in1b.wait()
        lax.fori_loop(jnp.int32(n_it // 2), jnp.int32(n_it), body, jnp.int32(0))
        out1 = pltpu.async_copy(outb.at[pl.ds(ch, ch)],
                                out_hbm.at[pl.ds(2 * base + jnp.int32(ch), ch)],
                                sem_out)
        out0.wait()
        out1.wait()

    return sc_call


def kernel(measurements, tMP, tKF, idxMP, idxKF):
    m = measurements.shape[0]
    out_dtype = jnp.promote_types(tMP.dtype, tKF.dtype)
    kf = measurements[:, 0].astype(jnp.int32)
    mp = measurements[:, 1].astype(jnp.int32)
    tmpt = tMP.astype(jnp.float32).T.reshape(-1)
    tkf = tKF.astype(jnp.float32).reshape(-1)
    idxkf = jnp.concatenate(
        [idxKF.astype(jnp.int32),
         jnp.arange(_N_KF, _L, dtype=jnp.int32)])
    idxmp = idxMP.astype(jnp.int32)
    out = _build_sc_call(m)(kf, mp, tmpt, tkf, idxkf, idxmp)
    out = out.reshape(m // 128, 2, 128).transpose(0, 2, 1).reshape(m, 2)
    return out.astype(out_dtype)


# speedup vs baseline: 1.0438x; 1.0160x over previous
import functools

import jax
import jax.numpy as jnp
from jax import lax
from jax.experimental import pallas as pl
from jax.experimental.pallas import tpu as pltpu
from jax.experimental.pallas import tpu_sc as plsc

_N_KF = 8
_N_MP = 32
_FX = 320.0
_FY = 320.0
_CX = 320.0
_CY = 240.0
_L = 16


@functools.lru_cache(maxsize=None)
def _build_sc_call(m: int):
    info = plsc.get_sparse_core_info()
    nc, ns = info.num_cores, info.num_subcores
    nw = nc * ns
    assert m % (nw * _L) == 0
    ch = m // nw
    n_it = ch // _L
    hf = ch // 2

    mesh = plsc.VectorSubcoreMesh(core_axis_name="c", subcore_axis_name="s")

    @functools.partial(
        pl.kernel,
        out_type=jax.ShapeDtypeStruct((2 * m,), jnp.float32),
        mesh=mesh,
        scratch_types=[
            pltpu.VMEM((ch,), jnp.int32),
            pltpu.VMEM((ch,), jnp.int32),
            pltpu.VMEM((2 * ch,), jnp.float32),
            pltpu.VMEM((_N_KF * _N_MP,), jnp.float32),
            pltpu.VMEM((_N_KF * _N_MP,), jnp.float32),
            pltpu.VMEM((_L,), jnp.int32),
            pltpu.VMEM((_N_MP,), jnp.int32),
            pltpu.VMEM((3 * _N_MP,), jnp.float32),
            pltpu.VMEM((_N_KF * 16,), jnp.float32),
            pltpu.VMEM((_L,), jnp.int32),
            pltpu.VMEM((_N_MP,), jnp.int32),
            pltpu.SemaphoreType.DMA,
            pltpu.SemaphoreType.DMA,
            pltpu.SemaphoreType.DMA,
            pltpu.SemaphoreType.DMA,
            pltpu.SemaphoreType.DMA,
            pltpu.SemaphoreType.DMA,
        ],
        compiler_params=pltpu.CompilerParams(needs_layout_passes=False),
    )
    def sc_call(kf_hbm, mp_hbm, tmpt_hbm, tkf_hbm, idxkf_hbm, idxmp_hbm,
                out_hbm, inkf, inmp, outb, tblx, tbly,
                poskf, posmp, tmpt, tkf, idxkf, idxmp,
                sem_in0, sem_in1, sem_in2, sem_in3, sem_out, sem_tbl):
        wid = lax.axis_index("s") * nc + lax.axis_index("c")
        base = wid * jnp.int32(ch)
        sem_ins = (sem_in0, sem_in1, sem_in2, sem_in3)
        qtr = ch // 4

        tw = [pltpu.async_copy(s, d, sem_tbl)
              for s, d in ((tmpt_hbm, tmpt), (tkf_hbm, tkf),
                           (idxkf_hbm, idxkf), (idxmp_hbm, idxmp))]
        inw = []
        for q in range(4):
            o = q * qtr
            inw.append((
                pltpu.async_copy(kf_hbm.at[pl.ds(base + jnp.int32(o), qtr)],
                                 inkf.at[pl.ds(o, qtr)], sem_ins[q]),
                pltpu.async_copy(mp_hbm.at[pl.ds(base + jnp.int32(o), qtr)],
                                 inmp.at[pl.ds(o, qtr)], sem_ins[q])))
        for w in tw:
            w.wait()

        iota = lax.iota(jnp.int32, _L)

        plsc.store_scatter(poskf, [idxkf[...]], iota)
        plsc.store_scatter(posmp, [idxmp[pl.ds(0, _L)]], iota)
        plsc.store_scatter(posmp, [idxmp[pl.ds(_L, _L)]], iota + _L)

        xp, yp, zp = [], [], []
        for h in range(2):
            pos = posmp[pl.ds(h * _L, _L)]
            xp.append(plsc.load_gather(tmpt, [pos]))
            yp.append(plsc.load_gather(tmpt, [pos + _N_MP]))
            zp.append(plsc.load_gather(tmpt, [pos + 2 * _N_MP]))

        pk = poskf[...]
        for v in range(_N_KF):
            row = tkf[pl.ds(pk[v] * jnp.int32(16), 16)]
            a00 = row[0]; a01 = row[1]; a02 = row[2]; a03 = row[3]
            a10 = row[4]; a11 = row[5]; a12 = row[6]; a13 = row[7]
            a20 = row[8]; a21 = row[9]; a22 = row[10]; a23 = row[11]
            for h in range(2):
                r0 = a00 * xp[h] + a01 * yp[h] + a02 * zp[h] + a03
                r1 = a10 * xp[h] + a11 * yp[h] + a12 * zp[h] + a13
                r2 = a20 * xp[h] + a21 * yp[h] + a22 * zp[h] + a23
                inv = 1.0 / r2
                tb = v * _N_MP + h * _L
                tblx[pl.ds(tb, _L)] = r0 * inv * _FX + _CX
                tbly[pl.ds(tb, _L)] = r1 * inv * _FY + _CY

        def body(i, carry):
            off = i * jnp.int32(_L)
            off2 = (jnp.right_shift(i, 3) * jnp.int32(256)
                    + jnp.bitwise_and(i, 7) * jnp.int32(_L))
            idx = jnp.left_shift(inkf[pl.ds(off, _L)], 5) + inmp[pl.ds(off, _L)]
            outb[pl.ds(off2, _L)] = plsc.load_gather(tblx, [idx])
            outb[pl.ds(off2 + jnp.int32(128), _L)] = plsc.load_gather(tbly, [idx])
            return carry

        nq = n_it // 4
        outw = []
        for q in range(4):
            inw[q][0].wait()
            inw[q][1].wait()
            lax.fori_loop(jnp.int32(q * nq), jnp.int32((q + 1) * nq),
                          body, jnp.int32(0))
            o2 = 2 * q * qtr
            outw.append(pltpu.async_copy(
                outb.at[pl.ds(o2, 2 * qtr)],
                out_hbm.at[pl.ds(2 * base + jnp.int32(o2), 2 * qtr)], sem_out))
        for w in outw:
            w.wait()

    return sc_call


def kernel(measurements, tMP, tKF, idxMP, idxKF):
    m = measurements.shape[0]
    out_dtype = jnp.promote_types(tMP.dtype, tKF.dtype)
    kf = measurements[:, 0].astype(jnp.int32)
    mp = measurements[:, 1].astype(jnp.int32)
    tmpt = tMP.astype(jnp.float32).T.reshape(-1)
    tkf = tKF.astype(jnp.float32).reshape(-1)
    idxkf = jnp.concatenate(
        [idxKF.astype(jnp.int32),
         jnp.arange(_N_KF, _L, dtype=jnp.int32)])
    idxmp = idxMP.astype(jnp.int32)
    out = _build_sc_call(m)(kf, mp, tmpt, tkf, idxkf, idxmp)
    out = out.reshape(m // 128, 2, 128).transpose(0, 2, 1).reshape(m, 2)
    return out.astype(out_dtype)


# speedup vs baseline: 1.0458x; 1.0019x over previous
import functools

import jax
import jax.numpy as jnp
from jax import lax
from jax.experimental import pallas as pl
from jax.experimental.pallas import tpu as pltpu
from jax.experimental.pallas import tpu_sc as plsc

_N_KF = 8
_N_MP = 32
_FX = 320.0
_FY = 320.0
_CX = 320.0
_CY = 240.0
_L = 16


@functools.lru_cache(maxsize=None)
def _build_sc_call(m: int):
    info = plsc.get_sparse_core_info()
    nc, ns = info.num_cores, info.num_subcores
    nw = nc * ns
    assert m % (nw * _L) == 0
    ch = m // nw
    n_it = ch // _L
    hf = ch // 2

    mesh = plsc.VectorSubcoreMesh(core_axis_name="c", subcore_axis_name="s")

    @functools.partial(
        pl.kernel,
        out_type=jax.ShapeDtypeStruct((2 * m,), jnp.float32),
        mesh=mesh,
        scratch_types=[
            pltpu.VMEM((ch,), jnp.int32),
            pltpu.VMEM((ch,), jnp.int32),
            pltpu.VMEM((2 * ch,), jnp.float32),
            pltpu.VMEM((_N_KF * _N_MP,), jnp.float32),
            pltpu.VMEM((_N_KF * _N_MP,), jnp.float32),
            pltpu.VMEM((_L,), jnp.int32),
            pltpu.VMEM((_N_MP,), jnp.int32),
            pltpu.VMEM((3 * _N_MP,), jnp.float32),
            pltpu.VMEM((_N_KF * 16,), jnp.float32),
            pltpu.VMEM((_L,), jnp.int32),
            pltpu.VMEM((_N_MP,), jnp.int32),
            pltpu.SemaphoreType.DMA,
            pltpu.SemaphoreType.DMA,
            pltpu.SemaphoreType.DMA,
            pltpu.SemaphoreType.DMA,
            pltpu.SemaphoreType.DMA,
            pltpu.SemaphoreType.DMA,
        ],
        compiler_params=pltpu.CompilerParams(needs_layout_passes=False),
    )
    def sc_call(kf_hbm, mp_hbm, tmpt_hbm, tkf_hbm, idxkf_hbm, idxmp_hbm,
                out_hbm, inkf, inmp, outb, tblx, tbly,
                poskf, posmp, tmpt, tkf, idxkf, idxmp,
                sem_in0, sem_in1, sem_in2, sem_in3, sem_out, sem_tbl):
        wid = lax.axis_index("s") * nc + lax.axis_index("c")
        base = wid * jnp.int32(ch)
        sem_ins = (sem_in0, sem_in1, sem_in2, sem_in3)
        qtr = ch // 4

        tw = [pltpu.async_copy(s, d, sem_tbl)
              for s, d in ((tmpt_hbm, tmpt), (tkf_hbm, tkf),
                           (idxkf_hbm, idxkf), (idxmp_hbm, idxmp))]
        inw = []
        for q in range(4):
            o = q * qtr
            inw.append((
                pltpu.async_copy(kf_hbm.at[pl.ds(base + jnp.int32(o), qtr)],
                                 inkf.at[pl.ds(o, qtr)], sem_ins[q]),
                pltpu.async_copy(mp_hbm.at[pl.ds(base + jnp.int32(o), qtr)],
                                 inmp.at[pl.ds(o, qtr)], sem_ins[q])))
        for w in tw:
            w.wait()

        iota = lax.iota(jnp.int32, _L)

        plsc.store_scatter(poskf, [idxkf[...]], iota)
        plsc.store_scatter(posmp, [idxmp[pl.ds(0, _L)]], iota)
        plsc.store_scatter(posmp, [idxmp[pl.ds(_L, _L)]], iota + _L)

        xp, yp, zp = [], [], []
        for h in range(2):
            pos = posmp[pl.ds(h * _L, _L)]
            xp.append(plsc.load_gather(tmpt, [pos]))
            yp.append(plsc.load_gather(tmpt, [pos + _N_MP]))
            zp.append(plsc.load_gather(tmpt, [pos + 2 * _N_MP]))

        pk = poskf[...]
        for v in range(_N_KF):
            row = tkf[pl.ds(pk[v] * jnp.int32(16), 16)]
            a00 = row[0]; a01 = row[1]; a02 = row[2]; a03 = row[3]
            a10 = row[4]; a11 = row[5]; a12 = row[6]; a13 = row[7]
            a20 = row[8]; a21 = row[9]; a22 = row[10]; a23 = row[11]
            for h in range(2):
                r0 = a00 * xp[h] + a01 * yp[h] + a02 * zp[h] + a03
                r1 = a10 * xp[h] + a11 * yp[h] + a12 * zp[h] + a13
                r2 = a20 * xp[h] + a21 * yp[h] + a22 * zp[h] + a23
                inv = 1.0 / r2
                tb = v * _N_MP + h * _L
                tblx[pl.ds(tb, _L)] = r0 * inv * _FX + _CX
                tbly[pl.ds(tb, _L)] = r1 * inv * _FY + _CY

        def body(i, carry):
            for u in range(2):
                k = jnp.int32(2) * i + jnp.int32(u)
                off = k * jnp.int32(_L)
                off2 = (jnp.right_shift(k, 3) * jnp.int32(256)
                        + jnp.bitwise_and(k, 7) * jnp.int32(_L))
                idx = (jnp.left_shift(inkf[pl.ds(off, _L)], 5)
                       + inmp[pl.ds(off, _L)])
                outb[pl.ds(off2, _L)] = plsc.load_gather(tblx, [idx])
                outb[pl.ds(off2 + jnp.int32(128), _L)] = plsc.load_gather(tbly, [idx])
            return carry

        nq = n_it // 4
        outw = []
        for q in range(4):
            inw[q][0].wait()
            inw[q][1].wait()
            lax.fori_loop(jnp.int32(q * nq // 2), jnp.int32((q + 1) * nq // 2),
                          body, jnp.int32(0))
            o2 = 2 * q * qtr
            outw.append(pltpu.async_copy(
                outb.at[pl.ds(o2, 2 * qtr)],
                out_hbm.at[pl.ds(2 * base + jnp.int32(o2), 2 * qtr)], sem_out))
        for w in outw:
            w.wait()

    return sc_call


def kernel(measurements, tMP, tKF, idxMP, idxKF):
    m = measurements.shape[0]
    out_dtype = jnp.promote_types(tMP.dtype, tKF.dtype)
    kf = measurements[:, 0].astype(jnp.int32)
    mp = measurements[:, 1].astype(jnp.int32)
    tmpt = tMP.astype(jnp.float32).T.reshape(-1)
    tkf = tKF.astype(jnp.float32).reshape(-1)
    idxkf = jnp.concatenate(
        [idxKF.astype(jnp.int32),
         jnp.arange(_N_KF, _L, dtype=jnp.int32)])
    idxmp = idxMP.astype(jnp.int32)
    out = _build_sc_call(m)(kf, mp, tmpt, tkf, idxkf, idxmp)
    out = out.reshape(m // 128, 2, 128).transpose(0, 2, 1).reshape(m, 2)
    return out.astype(out_dtype)
